# baseline (speedup 1.0000x reference)
import functools

import jax
import jax.numpy as jnp
from jax import lax
from jax.experimental import pallas as pl
from jax.experimental.pallas import tpu as pltpu
from jax.experimental.pallas import tpu_sc as plsc

_BATCH = 16384
_D = 64
_NC = 2
_NS = 16
_NW = _NC * _NS
_BPW = _BATCH // _NW
_CHUNK = 128
_NCHUNK = _BPW // _CHUNK

_BLK = 2048


def _gather_body(uid_hbm, iid_hbm, uemb_hbm, iemb_hbm, uout_hbm, iout_hbm,
                 uidx_v, iidx_v, urows_v, irows_v, usem, isem):
    wid = lax.axis_index("s") * _NC + lax.axis_index("c")
    base = wid * _BPW
    pltpu.sync_copy(uid_hbm.at[pl.ds(base, _BPW)], uidx_v)
    pltpu.sync_copy(iid_hbm.at[pl.ds(base, _BPW)], iidx_v)
    for k in range(_NCHUNK):
        sl = pl.ds(k * _CHUNK, _CHUNK)
        pltpu.async_copy(uemb_hbm.at[uidx_v.at[sl]], urows_v.at[sl], usem)
        pltpu.async_copy(iemb_hbm.at[iidx_v.at[sl]], irows_v.at[sl], isem)
    for k in range(_NCHUNK):
        sl = pl.ds(k * _CHUNK, _CHUNK)
        pltpu.make_async_copy(uemb_hbm.at[uidx_v.at[sl]], urows_v.at[sl], usem).wait()
        pltpu.make_async_copy(iemb_hbm.at[iidx_v.at[sl]], irows_v.at[sl], isem).wait()
    pltpu.sync_copy(urows_v, uout_hbm.at[pl.ds(base, _BPW)])
    pltpu.sync_copy(irows_v, iout_hbm.at[pl.ds(base, _BPW)])


def _mlp_body(u_ref, i_ref, w1u_ref, w1i_ref, b1_ref, w2_ref, b2_ref,
              w3_ref, b3_ref, out_ref):
    h = jnp.dot(u_ref[...], w1u_ref[...], preferred_element_type=jnp.float32)
    h = h + jnp.dot(i_ref[...], w1i_ref[...], preferred_element_type=jnp.float32)
    h = jnp.maximum(h + b1_ref[...], 0.0)
    h = jnp.dot(h, w2_ref[...], preferred_element_type=jnp.float32) + b2_ref[...]
    h = jnp.maximum(h, 0.0)
    logit = jnp.dot(h, w3_ref[...], preferred_element_type=jnp.float32) + b3_ref[...]
    out_ref[...] = jax.nn.sigmoid(logit)


@jax.jit
def kernel(user_ids, item_ids, user_emb, item_emb, W1, b1, W2, b2, W3, b3):
    mesh = plsc.VectorSubcoreMesh(core_axis_name="c", subcore_axis_name="s",
                                  num_cores=_NC, num_subcores=_NS)
    gather = pl.kernel(
        _gather_body,
        out_type=(
            jax.ShapeDtypeStruct((_BATCH, _D), jnp.float32),
            jax.ShapeDtypeStruct((_BATCH, _D), jnp.float32),
        ),
        mesh=mesh,
        scratch_types=[
            pltpu.VMEM((_BPW,), jnp.int32),
            pltpu.VMEM((_BPW,), jnp.int32),
            pltpu.VMEM((_BPW, _D), jnp.float32),
            pltpu.VMEM((_BPW, _D), jnp.float32),
            pltpu.SemaphoreType.DMA,
            pltpu.SemaphoreType.DMA,
        ],
        compiler_params=pltpu.CompilerParams(use_tc_tiling_on_sc=False),
    )
    u_vecs, i_vecs = gather(user_ids, item_ids, user_emb, item_emb)

    w1u = W1[:_D]
    w1i = W1[_D:]
    b1r = b1.reshape(1, -1)
    b2r = b2.reshape(1, -1)
    b3r = b3.reshape(1, 1)

    grid = _BATCH // _BLK
    out = pl.pallas_call(
        _mlp_body,
        grid=(grid,),
        in_specs=[
            pl.BlockSpec((_BLK, _D), lambda j: (j, 0)),
            pl.BlockSpec((_BLK, _D), lambda j: (j, 0)),
            pl.BlockSpec((_D, 128), lambda j: (0, 0)),
            pl.BlockSpec((_D, 128), lambda j: (0, 0)),
            pl.BlockSpec((1, 128), lambda j: (0, 0)),
            pl.BlockSpec((128, _D), lambda j: (0, 0)),
            pl.BlockSpec((1, _D), lambda j: (0, 0)),
            pl.BlockSpec((_D, 1), lambda j: (0, 0)),
            pl.BlockSpec((1, 1), lambda j: (0, 0)),
        ],
        out_specs=pl.BlockSpec((_BLK, 1), lambda j: (j, 0)),
        out_shape=jax.ShapeDtypeStruct((_BATCH, 1), jnp.float32),
    )(u_vecs, i_vecs, w1u, w1i, b1r, W2, b2r, W3, b3r)
    return out.reshape(_BATCH)


# speedup vs baseline: 2.0002x; 2.0002x over previous
import jax
import jax.numpy as jnp
from jax import lax
from jax.experimental import pallas as pl
from jax.experimental.pallas import tpu as pltpu
from jax.experimental.pallas import tpu_sc as plsc

_BATCH = 16384
_D = 64
_ROWS = 1000000
_R = 4096
_NSTEP = 123
_SPLIT = _R * _NSTEP

_NC = 2
_NS = 16
_NW = _NC * _NS
_BPW = _BATCH // _NW
_CHUNK = 128
_NCHUNK = _BPW // _CHUNK

_BLK = 2048


def _pack_body(a_ref, b_ref, out_ref):
    out_ref[:, 0:_D] = jnp.transpose(a_ref[...])
    out_ref[:, _D:128] = jnp.transpose(b_ref[...])


def _pack(embT):
    return pl.pallas_call(
        _pack_body,
        grid=(_NSTEP,),
        in_specs=[
            pl.BlockSpec((_D, _R), lambda j: (0, j)),
            pl.BlockSpec((_D, _R),
                         lambda j: (0, jnp.minimum(j + _NSTEP, _ROWS // _R))),
        ],
        out_specs=pl.BlockSpec((_R, 128), lambda j: (j, 0)),
        out_shape=jax.ShapeDtypeStruct((_SPLIT, 128), jnp.float32),
    )(embT, embT)


def _gather_body(ug_hbm, ig_hbm, upk_hbm, ipk_hbm, uout_hbm, iout_hbm,
                 uidx_v, iidx_v, ubuf, ibuf, usem, isem):
    wid = lax.axis_index("s") * _NC + lax.axis_index("c")
    base = wid * _BPW
    pltpu.sync_copy(ug_hbm.at[pl.ds(base, _BPW)], uidx_v)
    pltpu.sync_copy(ig_hbm.at[pl.ds(base, _BPW)], iidx_v)

    def fire(c, slot):
        csl = pl.ds(c * _CHUNK, _CHUNK)
        cu = pltpu.async_copy(upk_hbm.at[uidx_v.at[csl]], ubuf.at[slot], usem)
        ci = pltpu.async_copy(ipk_hbm.at[iidx_v.at[csl]], ibuf.at[slot], isem)
        return cu, ci

    cu, ci = fire(0, 0)
    for c in range(_NCHUNK):
        slot = c % 2
        cu.wait()
        ci.wait()
        nxt = fire(c + 1, 1 - slot) if c + 1 < _NCHUNK else None
        osl = pl.ds(base + c * _CHUNK, _CHUNK)
        pltpu.sync_copy(ubuf.at[slot], uout_hbm.at[osl])
        pltpu.sync_copy(ibuf.at[slot], iout_hbm.at[osl])
        if nxt is not None:
            cu, ci = nxt


def _mlp_body(ur_ref, ir_ref, uid_ref, iid_ref, w1u_ref, w1i_ref, b1_ref,
              w2_ref, b2_ref, w3_ref, b3_ref, out_ref):
    usel = uid_ref[...] < _SPLIT
    isel = iid_ref[...] < _SPLIT
    u = jnp.where(usel, ur_ref[:, 0:_D], ur_ref[:, _D:128])
    i = jnp.where(isel, ir_ref[:, 0:_D], ir_ref[:, _D:128])
    h = jnp.dot(u, w1u_ref[...], preferred_element_type=jnp.float32)
    h = h + jnp.dot(i, w1i_ref[...], preferred_element_type=jnp.float32)
    h = jnp.maximum(h + b1_ref[...], 0.0)
    h = jnp.dot(h, w2_ref[...], preferred_element_type=jnp.float32) + b2_ref[...]
    h = jnp.maximum(h, 0.0)
    logit = jnp.dot(h, w3_ref[...], preferred_element_type=jnp.float32) + b3_ref[...]
    out_ref[...] = jax.nn.sigmoid(logit)


@jax.jit
def kernel(user_ids, item_ids, user_emb, item_emb, W1, b1, W2, b2, W3, b3):
    upk = _pack(user_emb.T)
    ipk = _pack(item_emb.T)

    ug = jnp.where(user_ids < _SPLIT, user_ids, user_ids - _SPLIT)
    ig = jnp.where(item_ids < _SPLIT, item_ids, item_ids - _SPLIT)

    mesh = plsc.VectorSubcoreMesh(core_axis_name="c", subcore_axis_name="s",
                                  num_cores=_NC, num_subcores=_NS)
    gather = pl.kernel(
        _gather_body,
        out_type=(
            jax.ShapeDtypeStruct((_BATCH, 128), jnp.float32),
            jax.ShapeDtypeStruct((_BATCH, 128), jnp.float32),
        ),
        mesh=mesh,
        scratch_types=[
            pltpu.VMEM((_BPW,), jnp.int32),
            pltpu.VMEM((_BPW,), jnp.int32),
            pltpu.VMEM((2, _CHUNK, 128), jnp.float32),
            pltpu.VMEM((2, _CHUNK, 128), jnp.float32),
            pltpu.SemaphoreType.DMA,
            pltpu.SemaphoreType.DMA,
        ],
    )
    urows, irows = gather(ug, ig, upk, ipk)

    w1u = W1[:_D]
    w1i = W1[_D:]
    b1r = b1.reshape(1, -1)
    b2r = b2.reshape(1, -1)
    b3r = b3.reshape(1, 1)
    uid2 = user_ids.reshape(_BATCH, 1)
    iid2 = item_ids.reshape(_BATCH, 1)

    grid = _BATCH // _BLK
    out = pl.pallas_call(
        _mlp_body,
        grid=(grid,),
        in_specs=[
            pl.BlockSpec((_BLK, 128), lambda j: (j, 0)),
            pl.BlockSpec((_BLK, 128), lambda j: (j, 0)),
            pl.BlockSpec((_BLK, 1), lambda j: (j, 0)),
            pl.BlockSpec((_BLK, 1), lambda j: (j, 0)),
            pl.BlockSpec((_D, 128), lambda j: (0, 0)),
            pl.BlockSpec((_D, 128), lambda j: (0, 0)),
            pl.BlockSpec((1, 128), lambda j: (0, 0)),
            pl.BlockSpec((128, _D), lambda j: (0, 0)),
            pl.BlockSpec((1, _D), lambda j: (0, 0)),
            pl.BlockSpec((_D, 1), lambda j: (0, 0)),
            pl.BlockSpec((1, 1), lambda j: (0, 0)),
        ],
        out_specs=pl.BlockSpec((_BLK, 1), lambda j: (j, 0)),
        out_shape=jax.ShapeDtypeStruct((_BATCH, 1), jnp.float32),
    )(urows, irows, uid2, iid2, w1u, w1i, b1r, W2, b2r, W3, b3r)
    return out.reshape(_BATCH)


# speedup vs baseline: 2.2530x; 1.1264x over previous
import jax
import jax.numpy as jnp
from jax import lax
from jax.experimental import pallas as pl
from jax.experimental.pallas import tpu as pltpu
from jax.experimental.pallas import tpu_sc as plsc

_BATCH = 16384
_D = 64
_ROWS = 1000000
_R = 8192
_NSTEP = 62
_SPLIT = _R * _NSTEP

_NC = 2
_NS = 16
_NW = _NC * _NS
_BPW = _BATCH // _NW
_CHUNK = 128
_NCHUNK = _BPW // _CHUNK

_BLK = 2048


def _pack_body(a_ref, b_ref, out_ref):
    out_ref[:, 0:_D] = jnp.transpose(a_ref[...])
    out_ref[:, _D:128] = jnp.transpose(b_ref[...])


def _pack(embT):
    return pl.pallas_call(
        _pack_body,
        grid=(_NSTEP,),
        in_specs=[
            pl.BlockSpec((_D, _R), lambda j: (0, j)),
            pl.BlockSpec((_D, _R),
                         lambda j: (0, jnp.minimum(j + _NSTEP, _ROWS // _R))),
        ],
        out_specs=pl.BlockSpec((_R, 128), lambda j: (j, 0)),
        out_shape=jax.ShapeDtypeStruct((_SPLIT, 128), jnp.float32),
    )(embT, embT)


def _gather_body(ug_hbm, ig_hbm, upk_hbm, ipk_hbm, uout_hbm, iout_hbm,
                 uidx_v, iidx_v, ubuf, ibuf, usem, isem):
    wid = lax.axis_index("s") * _NC + lax.axis_index("c")
    base = wid * _BPW
    pltpu.sync_copy(ug_hbm.at[pl.ds(base, _BPW)], uidx_v)
    pltpu.sync_copy(ig_hbm.at[pl.ds(base, _BPW)], iidx_v)

    def fire(c, slot):
        csl = pl.ds(c * _CHUNK, _CHUNK)
        cu = pltpu.async_copy(upk_hbm.at[uidx_v.at[csl]], ubuf.at[slot], usem)
        ci = pltpu.async_copy(ipk_hbm.at[iidx_v.at[csl]], ibuf.at[slot], isem)
        return cu, ci

    cu, ci = fire(0, 0)
    for c in range(_NCHUNK):
        slot = c % 2
        cu.wait()
        ci.wait()
        nxt = fire(c + 1, 1 - slot) if c + 1 < _NCHUNK else None
        osl = pl.ds(base + c * _CHUNK, _CHUNK)
        pltpu.sync_copy(ubuf.at[slot], uout_hbm.at[osl])
        pltpu.sync_copy(ibuf.at[slot], iout_hbm.at[osl])
        if nxt is not None:
            cu, ci = nxt


def _mlp_body(ur_ref, ir_ref, uid_ref, iid_ref, w1u_ref, w1i_ref, b1_ref,
              w2_ref, b2_ref, w3_ref, b3_ref, out_ref):
    usel = uid_ref[...] < _SPLIT
    isel = iid_ref[...] < _SPLIT
    u = jnp.where(usel, ur_ref[:, 0:_D], ur_ref[:, _D:128])
    i = jnp.where(isel, ir_ref[:, 0:_D], ir_ref[:, _D:128])
    h = jnp.dot(u, w1u_ref[...], preferred_element_type=jnp.float32)
    h = h + jnp.dot(i, w1i_ref[...], preferred_element_type=jnp.float32)
    h = jnp.maximum(h + b1_ref[...], 0.0)
    h = jnp.dot(h, w2_ref[...], preferred_element_type=jnp.float32) + b2_ref[...]
    h = jnp.maximum(h, 0.0)
    logit = jnp.dot(h, w3_ref[...], preferred_element_type=jnp.float32) + b3_ref[...]
    out_ref[...] = jax.nn.sigmoid(logit)


@jax.jit
def kernel(user_ids, item_ids, user_emb, item_emb, W1, b1, W2, b2, W3, b3):
    upk = _pack(user_emb.T)
    ipk = _pack(item_emb.T)

    ug = jnp.where(user_ids < _SPLIT, user_ids, user_ids - _SPLIT)
    ig = jnp.where(item_ids < _SPLIT, item_ids, item_ids - _SPLIT)

    mesh = plsc.VectorSubcoreMesh(core_axis_name="c", subcore_axis_name="s",
                                  num_cores=_NC, num_subcores=_NS)
    gather = pl.kernel(
        _gather_body,
        out_type=(
            jax.ShapeDtypeStruct((_BATCH, 128), jnp.float32),
            jax.ShapeDtypeStruct((_BATCH, 128), jnp.float32),
        ),
        mesh=mesh,
        scratch_types=[
            pltpu.VMEM((_BPW,), jnp.int32),
            pltpu.VMEM((_BPW,), jnp.int32),
            pltpu.VMEM((2, _CHUNK, 128), jnp.float32),
            pltpu.VMEM((2, _CHUNK, 128), jnp.float32),
            pltpu.SemaphoreType.DMA,
            pltpu.SemaphoreType.DMA,
        ],
    )
    urows, irows = gather(ug, ig, upk, ipk)

    w1u = W1[:_D]
    w1i = W1[_D:]
    b1r = b1.reshape(1, -1)
    b2r = b2.reshape(1, -1)
    b3r = b3.reshape(1, 1)
    uid2 = user_ids.reshape(_BATCH, 1)
    iid2 = item_ids.reshape(_BATCH, 1)

    grid = _BATCH // _BLK
    out = pl.pallas_call(
        _mlp_body,
        grid=(grid,),
        in_specs=[
            pl.BlockSpec((_BLK, 128), lambda j: (j, 0)),
            pl.BlockSpec((_BLK, 128), lambda j: (j, 0)),
            pl.BlockSpec((_BLK, 1), lambda j: (j, 0)),
            pl.BlockSpec((_BLK, 1), lambda j: (j, 0)),
            pl.BlockSpec((_D, 128), lambda j: (0, 0)),
            pl.BlockSpec((_D, 128), lambda j: (0, 0)),
            pl.BlockSpec((1, 128), lambda j: (0, 0)),
            pl.BlockSpec((128, _D), lambda j: (0, 0)),
            pl.BlockSpec((1, _D), lambda j: (0, 0)),
            pl.BlockSpec((_D, 1), lambda j: (0, 0)),
            pl.BlockSpec((1, 1), lambda j: (0, 0)),
        ],
        out_specs=pl.BlockSpec((_BLK, 1), lambda j: (j, 0)),
        out_shape=jax.ShapeDtypeStruct((_BATCH, 1), jnp.float32),
    )(urows, irows, uid2, iid2, w1u, w1i, b1r, W2, b2r, W3, b3r)
    return out.reshape(_BATCH)


# speedup vs baseline: 2.5961x; 1.1522x over previous
import jax
import jax.numpy as jnp
from jax import lax
from jax.experimental import pallas as pl
from jax.experimental.pallas import tpu as pltpu
from jax.experimental.pallas import tpu_sc as plsc

_BATCH = 16384
_D = 64
_ROWS = 1000000
_R = 8192
_NSTEP = 31
_SPLIT = _R * _NSTEP
_LASTBLK = _ROWS // _R

_NC = 2
_NS = 16
_NW = _NC * _NS
_BPW = _BATCH // _NW
_CHUNK = 128
_NCHUNK = _BPW // _CHUNK

_BLK = 2048


def _bf16_hi(x):
    u = lax.bitcast_convert_type(x, jnp.uint32)
    round_bias = jnp.uint32(0x7FFF) + ((u >> 16) & jnp.uint32(1))
    return (u + round_bias) & jnp.uint32(0xFFFF0000)


def _pack_body(a_ref, b_ref, c_ref, d_ref, out_ref):
    at = jnp.transpose(a_ref[...])
    bt = jnp.transpose(b_ref[...])
    ct = jnp.transpose(c_ref[...])
    dt = jnp.transpose(d_ref[...])
    w0 = _bf16_hi(at) | (_bf16_hi(bt) >> 16)
    w1 = _bf16_hi(ct) | (_bf16_hi(dt) >> 16)
    out_ref[...] = lax.bitcast_convert_type(
        jnp.concatenate([w0, w1], axis=1), jnp.int32)


def _pack(embT):
    def mk(k):
        return pl.BlockSpec(
            (_D, _R), lambda j: (0, jnp.minimum(j + k * _NSTEP, _LASTBLK)))
    return pl.pallas_call(
        _pack_body,
        grid=(_NSTEP,),
        in_specs=[mk(0), mk(1), mk(2), mk(3)],
        out_specs=pl.BlockSpec((_R, 128), lambda j: (j, 0)),
        out_shape=jax.ShapeDtypeStruct((_SPLIT, 128), jnp.int32),
    )(embT, embT, embT, embT)


def _gather_body(ug_hbm, ig_hbm, upk_hbm, ipk_hbm, uout_hbm, iout_hbm,
                 uidx_v, iidx_v, ubuf, ibuf, usem, isem):
    wid = lax.axis_index("s") * _NC + lax.axis_index("c")
    base = wid * _BPW
    pltpu.sync_copy(ug_hbm.at[pl.ds(base, _BPW)], uidx_v)
    pltpu.sync_copy(ig_hbm.at[pl.ds(base, _BPW)], iidx_v)

    def fire(c, slot):
        csl = pl.ds(c * _CHUNK, _CHUNK)
        cu = pltpu.async_copy(upk_hbm.at[uidx_v.at[csl]], ubuf.at[slot], usem)
        ci = pltpu.async_copy(ipk_hbm.at[iidx_v.at[csl]], ibuf.at[slot], isem)
        return cu, ci

    cu, ci = fire(0, 0)
    for c in range(_NCHUNK):
        slot = c % 2
        cu.wait()
        ci.wait()
        nxt = fire(c + 1, 1 - slot) if c + 1 < _NCHUNK else None
        osl = pl.ds(base + c * _CHUNK, _CHUNK)
        pltpu.sync_copy(ubuf.at[slot], uout_hbm.at[osl])
        pltpu.sync_copy(ibuf.at[slot], iout_hbm.at[osl])
        if nxt is not None:
            cu, ci = nxt


def _unpack_rows(rows_i32, ids):
    w = lax.bitcast_convert_type(rows_i32, jnp.uint32)
    region = ids // _SPLIT
    half = jnp.where(region < 2, w[:, 0:_D], w[:, _D:128])
    bits = jnp.where((region % 2) == 0,
                     half & jnp.uint32(0xFFFF0000), half << 16)
    return lax.bitcast_convert_type(bits, jnp.float32)


def _mlp_body(ur_ref, ir_ref, uid_ref, iid_ref, w1u_ref, w1i_ref, b1_ref,
              w2_ref, b2_ref, w3_ref, b3_ref, out_ref):
    u = _unpack_rows(ur_ref[...], uid_ref[...])
    i = _unpack_rows(ir_ref[...], iid_ref[...])
    h = jnp.dot(u, w1u_ref[...], preferred_element_type=jnp.float32)
    h = h + jnp.dot(i, w1i_ref[...], preferred_element_type=jnp.float32)
    h = jnp.maximum(h + b1_ref[...], 0.0)
    h = jnp.dot(h, w2_ref[...], preferred_element_type=jnp.float32) + b2_ref[...]
    h = jnp.maximum(h, 0.0)
    logit = jnp.dot(h, w3_ref[...], preferred_element_type=jnp.float32) + b3_ref[...]
    out_ref[...] = jax.nn.sigmoid(logit)


@jax.jit
def kernel(user_ids, item_ids, user_emb, item_emb, W1, b1, W2, b2, W3, b3):
    upk = _pack(user_emb.T)
    ipk = _pack(item_emb.T)

    ug = user_ids % _SPLIT
    ig = item_ids % _SPLIT

    mesh = plsc.VectorSubcoreMesh(core_axis_name="c", subcore_axis_name="s",
                                  num_cores=_NC, num_subcores=_NS)
    gather = pl.kernel(
        _gather_body,
        out_type=(
            jax.ShapeDtypeStruct((_BATCH, 128), jnp.int32),
            jax.ShapeDtypeStruct((_BATCH, 128), jnp.int32),
        ),
        mesh=mesh,
        scratch_types=[
            pltpu.VMEM((_BPW,), jnp.int32),
            pltpu.VMEM((_BPW,), jnp.int32),
            pltpu.VMEM((2, _CHUNK, 128), jnp.int32),
            pltpu.VMEM((2, _CHUNK, 128), jnp.int32),
            pltpu.SemaphoreType.DMA,
            pltpu.SemaphoreType.DMA,
        ],
    )
    urows, irows = gather(ug, ig, upk, ipk)

    w1u = W1[:_D]
    w1i = W1[_D:]
    b1r = b1.reshape(1, -1)
    b2r = b2.reshape(1, -1)
    b3r = b3.reshape(1, 1)
    uid2 = user_ids.reshape(_BATCH, 1)
    iid2 = item_ids.reshape(_BATCH, 1)

    grid = _BATCH // _BLK
    out = pl.pallas_call(
        _mlp_body,
        grid=(grid,),
        in_specs=[
            pl.BlockSpec((_BLK, 128), lambda j: (j, 0)),
            pl.BlockSpec((_BLK, 128), lambda j: (j, 0)),
            pl.BlockSpec((_BLK, 1), lambda j: (j, 0)),
            pl.BlockSpec((_BLK, 1), lambda j: (j, 0)),
            pl.BlockSpec((_D, 128), lambda j: (0, 0)),
            pl.BlockSpec((_D, 128), lambda j: (0, 0)),
            pl.BlockSpec((1, 128), lambda j: (0, 0)),
            pl.BlockSpec((128, _D), lambda j: (0, 0)),
            pl.BlockSpec((1, _D), lambda j: (0, 0)),
            pl.BlockSpec((_D, 1), lambda j: (0, 0)),
            pl.BlockSpec((1, 1), lambda j: (0, 0)),
        ],
        out_specs=pl.BlockSpec((_BLK, 1), lambda j: (j, 0)),
        out_shape=jax.ShapeDtypeStruct((_BATCH, 1), jnp.float32),
    )(urows, irows, uid2, iid2, w1u, w1i, b1r, W2, b2r, W3, b3r)
    return out.reshape(_BATCH)
